# baseline (speedup 1.0000x reference)
import functools

import jax
import jax.numpy as jnp
from jax import lax
from jax.experimental import pallas as pl
from jax.experimental.pallas import tpu as pltpu
from jax.experimental.pallas import tpu_sc as plsc

N, C, H, W = 64, 3, 512, 512
L = 256
ROW = H * W
NROWS = N * C
NWORKERS = 32
ROWS_PER_W = NROWS // NWORKERS
CHUNK = 32768
CHUNKS_PER_ROW = ROW // CHUNK
NCHUNKS = ROWS_PER_W * CHUNKS_PER_ROW
NGROUPS = NCHUNKS // 3
LUT_STRIDE = L + 1
LUT_PAD = 784
VEC = 16


def _sc_body(img_hbm, lut_hbm, out_hbm, lut_a, lut_d,
             buf0, buf1, buf2, in_sem0, in_sem1, in_sem2,
             out_sem0, out_sem1, out_sem2):
    wid = lax.axis_index("s") * 2 + lax.axis_index("c")
    pltpu.sync_copy(lut_hbm.at[0], lut_a)
    pltpu.sync_copy(lut_hbm.at[1], lut_d)

    wbase = wid * ROWS_PER_W * ROW
    bufs = (buf0, buf1, buf2)
    in_sems = (in_sem0, in_sem1, in_sem2)
    out_sems = (out_sem0, out_sem1, out_sem2)

    def compute(buf, off_f):
        @plsc.parallel_loop(0, CHUNK // VEC, unroll=8)
        def vec_body(i):
            v = buf[pl.ds(i * VEC, VEC)]
            x = v * 255.0 + off_f
            x0 = x.astype(jnp.int32)
            w = x - x0.astype(jnp.float32)
            a = plsc.load_gather(lut_a, [x0])
            d = plsc.load_gather(lut_d, [x0])
            buf[pl.ds(i * VEC, VEC)] = a + w * d

    def fill(t, b):
        pltpu.async_copy(img_hbm.at[pl.ds(wbase + t * CHUNK, CHUNK)],
                         bufs[b], in_sems[b])

    def wait_fill(t, b):
        pltpu.make_async_copy(img_hbm.at[pl.ds(wbase + t * CHUNK, CHUNK)],
                              bufs[b], in_sems[b]).wait()

    def drain(t, b):
        pltpu.async_copy(bufs[b], out_hbm.at[pl.ds(wbase + t * CHUNK, CHUNK)],
                         out_sems[b])

    def wait_drain(t, b):
        pltpu.make_async_copy(bufs[b],
                              out_hbm.at[pl.ds(wbase + t * CHUNK, CHUNK)],
                              out_sems[b]).wait()

    fill(0, 0)
    fill(1, 1)

    def ring_body(g, _):
        t0 = 3 * g
        off = (lax.rem(wid * ROWS_PER_W + t0 // CHUNKS_PER_ROW, 3)
               * LUT_STRIDE).astype(jnp.float32)
        wait_fill(t0, 0)

        @pl.when(g >= 1)
        def _():
            wait_drain(t0 - 1, 2)
        fill(t0 + 2, 2)
        compute(bufs[0], off)
        drain(t0, 0)

        t1 = t0 + 1
        off = (lax.rem(wid * ROWS_PER_W + t1 // CHUNKS_PER_ROW, 3)
               * LUT_STRIDE).astype(jnp.float32)
        wait_fill(t1, 1)

        @pl.when(g <= NGROUPS - 2)
        def _():
            wait_drain(t1 - 1, 0)
            fill(t1 + 2, 0)
        compute(bufs[1], off)
        drain(t1, 1)

        t2 = t0 + 2
        off = (lax.rem(wid * ROWS_PER_W + t2 // CHUNKS_PER_ROW, 3)
               * LUT_STRIDE).astype(jnp.float32)
        wait_fill(t2, 2)

        @pl.when(g <= NGROUPS - 2)
        def _():
            wait_drain(t2 - 1, 1)
            fill(t2 + 2, 1)
        compute(bufs[2], off)
        drain(t2, 2)
        return 0

    lax.fori_loop(0, NGROUPS, ring_body, 0)
    wait_drain(NCHUNKS - 3, 0)
    wait_drain(NCHUNKS - 2, 1)
    wait_drain(NCHUNKS - 1, 2)


@jax.jit
def _lut_apply(img_flat, lut_flat):
    mesh = plsc.VectorSubcoreMesh(core_axis_name="c", subcore_axis_name="s")
    return pl.kernel(
        _sc_body,
        out_type=jax.ShapeDtypeStruct((N * C * ROW,), jnp.float32),
        mesh=mesh,
        scratch_types=[
            pltpu.VMEM((LUT_PAD,), jnp.float32),
            pltpu.VMEM((LUT_PAD,), jnp.float32),
            pltpu.VMEM((CHUNK,), jnp.float32),
            pltpu.VMEM((CHUNK,), jnp.float32),
            pltpu.VMEM((CHUNK,), jnp.float32),
            pltpu.SemaphoreType.DMA,
            pltpu.SemaphoreType.DMA,
            pltpu.SemaphoreType.DMA,
            pltpu.SemaphoreType.DMA,
            pltpu.SemaphoreType.DMA,
            pltpu.SemaphoreType.DMA,
        ],
        compiler_params=pltpu.CompilerParams(needs_layout_passes=False),
    )(img_flat, lut_flat)


def kernel(image, icrf):
    lut = jnp.concatenate([icrf, icrf[:, -1:]], axis=1).reshape(-1)
    a = jnp.pad(lut, (0, LUT_PAD - lut.shape[0]))
    d = jnp.pad(lut[1:] - lut[:-1], (0, LUT_PAD - lut.shape[0] + 1))
    out = _lut_apply(image.reshape(-1), jnp.stack([a, d]))
    return out.reshape(image.shape)
